# baseline (speedup 1.0000x reference)
import functools

import jax
import jax.numpy as jnp
from jax import lax
from jax.experimental import pallas as pl
from jax.experimental.pallas import tpu as pltpu
from jax.experimental.pallas import tpu_sc as plsc

N = 10000
E = 320000
H = 128
DE = 16
NC = 2
NS = 16
NW = NC * NS
EPW = E // NW
BE = 4000
BL = 4000

C = 40
NCHUNKS = EPW // C
MSG_NBUF = 5
MSG_OUTER = NCHUNKS // MSG_NBUF
MSG_TAIL = NCHUNKS - MSG_OUTER * MSG_NBUF
NBLK = N // C

C2 = 40
NCH2 = EPW // C2
MLP_NBUF = 5
MLP_OUTER = NCH2 // MLP_NBUF

_mesh = plsc.VectorSubcoreMesh(core_axis_name="c", subcore_axis_name="s")


def _prep_body(x_ref, wn_ref, bn_ref, we_ref, wcm_ref, be_ref, bm1_ref,
               h_ref, wc_ref, bc_ref):
    h_ref[...] = jnp.dot(x_ref[...], wn_ref[...],
                         preferred_element_type=jnp.float32) + bn_ref[...]
    wc = jnp.dot(we_ref[...], wcm_ref[...], preferred_element_type=jnp.float32)
    wc_ref[...] = wc
    bc_ref[...] = jnp.dot(be_ref[...], wcm_ref[...],
                          preferred_element_type=jnp.float32) + bm1_ref[...]


def _prep(x, Wn, bn, We, Wcm, be, bm1):
    return pl.pallas_call(
        _prep_body,
        out_shape=(
            jax.ShapeDtypeStruct((N, H), jnp.float32),
            jax.ShapeDtypeStruct((DE, H), jnp.float32),
            jax.ShapeDtypeStruct((1, H), jnp.float32),
        ),
    )(x, Wn, bn.reshape(1, H), We, Wcm, be.reshape(1, H), bm1.reshape(1, H))


def _edge_enc_body(ea_ref, w_ref, b_ref, e_ref):
    z = jnp.dot(ea_ref[...], w_ref[...],
                preferred_element_type=jnp.float32) + b_ref[...]
    u = jax.lax.bitcast_convert_type(z, jnp.uint32)
    b16 = (u + jnp.uint32(0x7FFF) + ((u >> 16) & jnp.uint32(1))) >> 16
    parts = []
    for g in range(H // 32):
        lo = b16[:, g * 32:g * 32 + 16]
        hi = b16[:, g * 32 + 16:g * 32 + 32]
        parts.append(lo | (hi << 16))
    e_ref[...] = jax.lax.bitcast_convert_type(
        jnp.concatenate(parts, axis=1), jnp.int32)


def _edge_enc(edge_attr, W, b):
    return pl.pallas_call(
        _edge_enc_body,
        grid=(E // BE,),
        in_specs=[
            pl.BlockSpec((BE, DE), lambda i: (i, 0)),
            pl.BlockSpec((DE, H), lambda i: (0, 0)),
            pl.BlockSpec((1, H), lambda i: (0, 0)),
        ],
        out_specs=pl.BlockSpec((BE, H // 2), lambda i: (i, 0)),
        out_shape=jax.ShapeDtypeStruct((E, H // 2), jnp.int32),
    )(edge_attr, W, b)


def _norm_mlp(p_ref, h_ref, w1_ref, b1_ref, w2_ref, b2_ref, lnw_ref, lnb_ref):
    agg = p_ref[0] + p_ref[1] + h_ref[...]
    z = jnp.maximum(jnp.dot(agg, w1_ref[...],
                            preferred_element_type=jnp.float32) + b1_ref[...], 0.0)
    z = jnp.dot(z, w2_ref[...], preferred_element_type=jnp.float32) + b2_ref[...]
    m = jnp.mean(z)
    v = jnp.mean((z - m) ** 2)
    z = (z - m) / (jnp.sqrt(v) + 1e-5)
    return jnp.maximum(z * lnw_ref[...] + lnb_ref[...], 0.0)


def _update_body(p_ref, h_ref, w1_ref, b1_ref, w2_ref, b2_ref, lnw_ref,
                 lnb_ref, o_ref):
    o_ref[...] = _norm_mlp(p_ref, h_ref, w1_ref, b1_ref, w2_ref, b2_ref,
                           lnw_ref, lnb_ref)


def _update(part, h, W1, b1, W2, b2, lnw, lnb):
    return pl.pallas_call(
        _update_body,
        out_shape=jax.ShapeDtypeStruct((N, H), jnp.float32),
    )(part, h, W1, b1.reshape(1, H), W2, b2.reshape(1, H),
      lnw.reshape(1, H), lnb.reshape(1, H))


def _final_update_body(p_ref, h_ref, w1_ref, b1_ref, w2_ref, b2_ref, lnw_ref,
                       lnb_ref, wa_ref, wb_ref, ha_ref, hb_ref):
    hn = _norm_mlp(p_ref, h_ref, w1_ref, b1_ref, w2_ref, b2_ref,
                   lnw_ref, lnb_ref)
    ha_ref[...] = jnp.dot(hn, wa_ref[...], preferred_element_type=jnp.float32)
    hb_ref[...] = jnp.dot(hn, wb_ref[...], preferred_element_type=jnp.float32)


def _final_update(part, h, W1, b1, W2, b2, lnw, lnb, WA, WB):
    return pl.pallas_call(
        _final_update_body,
        out_shape=(
            jax.ShapeDtypeStruct((N, H), jnp.float32),
            jax.ShapeDtypeStruct((N, H), jnp.float32),
        ),
    )(part, h, W1, b1.reshape(1, H), W2, b2.reshape(1, H),
      lnw.reshape(1, H), lnb.reshape(1, H), WA, WB)


def _lane_sum_body(t_ref, b2_ref, o_ref):
    j = lax.broadcasted_iota(jnp.int32, (128, 8), 0)
    g = lax.broadcasted_iota(jnp.int32, (128, 8), 1)
    m = (j // 16 == g).astype(jnp.float32)
    o_ref[...] = jnp.dot(t_ref[...], m,
                         preferred_element_type=jnp.float32) + b2_ref[0, 0]


def _lane_sum(t16_flat, bm2):
    t = t16_flat.reshape(E // 8, 128)
    out = pl.pallas_call(
        _lane_sum_body,
        grid=(E // 8 // BL,),
        in_specs=[
            pl.BlockSpec((BL, 128), lambda i: (i, 0)),
            pl.BlockSpec((1, 1), lambda i: (0, 0)),
        ],
        out_specs=pl.BlockSpec((BL, 8), lambda i: (i, 0)),
        out_shape=jax.ShapeDtypeStruct((E // 8, 8), jnp.float32),
    )(t, bm2.reshape(1, 1))
    return out.reshape(E)


@functools.partial(
    pl.kernel,
    out_type=jax.ShapeDtypeStruct((NC, N, H), jnp.float32),
    mesh=_mesh,
    compiler_params=pltpu.CompilerParams(needs_layout_passes=False),
    scratch_types=(
        [pltpu.VMEM((C,), jnp.int32) for _ in range(MSG_NBUF)]
        + [pltpu.VMEM((C,), jnp.int32) for _ in range(MSG_NBUF)]
        + [pltpu.VMEM((C, H), jnp.float32) for _ in range(MSG_NBUF)]
        + [pltpu.VMEM((C * H // 2,), jnp.int32) for _ in range(MSG_NBUF)]
        + [
            pltpu.VMEM_SHARED((N, H), jnp.float32),
            pltpu.SemaphoreType.DMA((MSG_NBUF,)),
            pltpu.SemaphoreType.DMA((MSG_NBUF,)),
            pltpu.SemaphoreType.DMA((MSG_NBUF,)),
            pltpu.SemaphoreType.DMA((MSG_NBUF,)),
            pltpu.SemaphoreType.DMA((MSG_NBUF,)),
        ]
    ),
)
def _sc_msg(h_hbm, e_hbm, src_hbm, dst_hbm, out_hbm, *refs):
    sidx = list(refs[0:MSG_NBUF])
    didx = list(refs[MSG_NBUF:2 * MSG_NBUF])
    gbuf = list(refs[2 * MSG_NBUF:3 * MSG_NBUF])
    ebuf = list(refs[3 * MSG_NBUF:4 * MSG_NBUF])
    agg, sem_si, sem_di, sem_g, sem_e, sem_sc = refs[4 * MSG_NBUF:]

    cid = lax.axis_index("c")
    sid = lax.axis_index("s")
    wid = cid * NS + sid
    base_w = wid * EPW

    zv = jnp.zeros((16,), jnp.float32)

    def zrow(r, carry):
        for k in range(H // 16):
            gbuf[0][r, pl.ds(k * 16, 16)] = zv
        return carry

    lax.fori_loop(0, C, zrow, 0)
    nblk = jnp.where(sid < NBLK - (NBLK // NS) * NS, NBLK // NS + 1, NBLK // NS)

    def zblk(j, carry):
        blk = sid + j * NS
        pltpu.sync_copy(gbuf[0], agg.at[pl.ds(blk * C, C)])
        return carry

    lax.fori_loop(0, nblk, zblk, 0)
    plsc.subcore_barrier()

    def compute_rows(gb, eb):
        def row4(r4, rc):
            for dr in range(4):
                r = r4 * 4 + dr
                for k in range(H // 32):
                    ev = eb[pl.ds(r * (H // 2) + k * 16, 16)]
                    ea_ = plsc.bitcast(ev << 16, jnp.float32)
                    eb_ = plsc.bitcast(ev & jnp.int32(-65536), jnp.float32)
                    g0 = gb[r, pl.ds(k * 32, 16)]
                    g1 = gb[r, pl.ds(k * 32 + 16, 16)]
                    gb[r, pl.ds(k * 32, 16)] = jnp.maximum(g0 + ea_, 0.0)
                    gb[r, pl.ds(k * 32 + 16, 16)] = jnp.maximum(g1 + eb_, 0.0)
            return rc

        lax.fori_loop(0, C // 4, row4, 0)

    def outer(o, carry):
        i0 = o * MSG_NBUF

        @pl.when(o > 0)
        def _drain():
            for b in range(MSG_NBUF):
                pltpu.make_async_copy(h_hbm.at[pl.ds(0, C)], gbuf[b],
                                      sem_sc.at[b]).wait()

        dsi, ddi, dg, de = [], [], [], []
        for b in range(MSG_NBUF):
            eb = base_w + (i0 + b) * C
            dsi.append(pltpu.async_copy(src_hbm.at[pl.ds(eb, C)], sidx[b],
                                        sem_si.at[b]))
            ddi.append(pltpu.async_copy(dst_hbm.at[pl.ds(eb, C)], didx[b],
                                        sem_di.at[b]))
        for b in range(MSG_NBUF):
            eb = base_w + (i0 + b) * C
            dsi[b].wait()
            dg.append(pltpu.async_copy(h_hbm.at[sidx[b]], gbuf[b],
                                       sem_g.at[b]))
            de.append(pltpu.async_copy(
                e_hbm.at[pl.ds(eb * (H // 2), C * H // 2)], ebuf[b],
                sem_e.at[b]))
        dsc = []
        for b in range(MSG_NBUF):
            dg[b].wait()
            de[b].wait()
            compute_rows(gbuf[b], ebuf[b])
            ddi[b].wait()
            dsc.append(pltpu.async_copy(gbuf[b], agg.at[didx[b]],
                                        sem_sc.at[b], add=True))
        return carry

    lax.fori_loop(0, MSG_OUTER, outer, 0)
    for b in range(MSG_NBUF):
        pltpu.make_async_copy(h_hbm.at[pl.ds(0, C)], gbuf[b],
                              sem_sc.at[b]).wait()

    plsc.subcore_barrier()

    def oblk(j, carry):
        blk = sid + j * NS
        pltpu.sync_copy(agg.at[pl.ds(blk * C, C)],
                        out_hbm.at[cid, pl.ds(blk * C, C)])
        return carry

    lax.fori_loop(0, nblk, oblk, 0)


@functools.partial(
    pl.kernel,
    out_type=jax.ShapeDtypeStruct((E * 16,), jnp.float32),
    mesh=_mesh,
    compiler_params=pltpu.CompilerParams(needs_layout_passes=False),
    scratch_types=(
        [
            pltpu.VMEM((EPW,), jnp.int32),
            pltpu.VMEM((EPW,), jnp.int32),
            pltpu.VMEM((H,), jnp.float32),
        ]
        + [pltpu.VMEM((C2, H), jnp.float32) for _ in range(MLP_NBUF)]
        + [pltpu.VMEM((C2, H), jnp.float32) for _ in range(MLP_NBUF)]
        + [pltpu.VMEM((C2 * H // 2,), jnp.int32) for _ in range(MLP_NBUF)]
        + [pltpu.VMEM((C2 * 16,), jnp.float32) for _ in range(MLP_NBUF)]
        + [
            pltpu.SemaphoreType.DMA((MLP_NBUF,)),
            pltpu.SemaphoreType.DMA((MLP_NBUF,)),
            pltpu.SemaphoreType.DMA((MLP_NBUF,)),
            pltpu.SemaphoreType.DMA((MLP_NBUF,)),
        ]
    ),
)
def _sc_edge_mlp(ha_hbm, hb_hbm, ec_hbm, src_hbm, dst_hbm, w2_hbm, out_hbm,
                 *refs):
    sidx_all, didx_all, wbuf = refs[0:3]
    abuf = list(refs[3:3 + MLP_NBUF])
    bbuf = list(refs[3 + MLP_NBUF:3 + 2 * MLP_NBUF])
    cbuf = list(refs[3 + 2 * MLP_NBUF:3 + 3 * MLP_NBUF])
    obuf = list(refs[3 + 3 * MLP_NBUF:3 + 4 * MLP_NBUF])
    sem_a, sem_b, sem_c, sem_o = refs[3 + 4 * MLP_NBUF:]

    cid = lax.axis_index("c")
    sid = lax.axis_index("s")
    wid = cid * NS + sid
    base_w = wid * EPW

    pltpu.sync_copy(w2_hbm, wbuf)
    pltpu.sync_copy(src_hbm.at[pl.ds(base_w, EPW)], sidx_all)
    pltpu.sync_copy(dst_hbm.at[pl.ds(base_w, EPW)], didx_all)
    wv = [wbuf[pl.ds(k * 16, 16)] for k in range(H // 16)]

    def outer(o, carry):
        i0 = o * MLP_NBUF
        da, db, dc = [], [], []
        for b in range(MLP_NBUF):
            i = i0 + b
            da.append(pltpu.async_copy(
                ha_hbm.at[sidx_all.at[pl.ds(i * C2, C2)]], abuf[b],
                sem_a.at[b]))
            db.append(pltpu.async_copy(
                hb_hbm.at[didx_all.at[pl.ds(i * C2, C2)]], bbuf[b],
                sem_b.at[b]))
            dc.append(pltpu.async_copy(
                ec_hbm.at[pl.ds((base_w + i * C2) * (H // 2), C2 * H // 2)],
                cbuf[b], sem_c.at[b]))
        for b in range(MLP_NBUF):
            i = i0 + b

            @pl.when(o > 0)
            def _drain(_b=b):
                pltpu.make_async_copy(out_hbm.at[pl.ds(0, C2 * 16)],
                                      obuf[_b], sem_o.at[_b]).wait()

            da[b].wait()
            db[b].wait()
            dc[b].wait()

            def row2(r2, rc, _a=abuf[b], _b=bbuf[b], _c=cbuf[b],
                     _o=obuf[b]):
                for dr in range(2):
                    r = r2 * 2 + dr
                    acc = None
                    for k in range(H // 32):
                        cv = _c[pl.ds(r * (H // 2) + k * 16, 16)]
                        c0 = plsc.bitcast(cv << 16, jnp.float32)
                        c1 = plsc.bitcast(cv & jnp.int32(-65536), jnp.float32)
                        t0 = (_a[r, pl.ds(k * 32, 16)]
                              + _b[r, pl.ds(k * 32, 16)] + c0)
                        t0 = jnp.maximum(t0, 0.0) * wv[2 * k]
                        t1 = (_a[r, pl.ds(k * 32 + 16, 16)]
                              + _b[r, pl.ds(k * 32 + 16, 16)] + c1)
                        t1 = jnp.maximum(t1, 0.0) * wv[2 * k + 1]
                        acc = t0 + t1 if acc is None else acc + t0 + t1
                    _o[pl.ds(r * 16, 16)] = acc
                return rc

            lax.fori_loop(0, C2 // 2, row2, 0)
            pltpu.async_copy(
                obuf[b], out_hbm.at[pl.ds((base_w + i * C2) * 16, C2 * 16)],
                sem_o.at[b])
        return carry

    lax.fori_loop(0, MLP_OUTER, outer, 0)
    for b in range(MLP_NBUF):
        pltpu.make_async_copy(out_hbm.at[pl.ds(0, C2 * 16)], obuf[b],
                              sem_o.at[b]).wait()


def kernel(x, edge_index, edge_attr, Wn, bn, We, be, gW1, gb1, gW2, gb2,
           ln_w, ln_b, Wm1, bm1, Wm2, bm2):
    src = edge_index[0]
    dst = edge_index[1]
    WA = Wm1[:H]
    WB = Wm1[H:2 * H]

    WCm = Wm1[2 * H:]

    h, WC, bC = _prep(x, Wn, bn, We, WCm, be, bm1)
    e = _edge_enc(edge_attr, We, be.reshape(1, H))

    e_flat = e.reshape(E * H // 2)

    for i in range(2):
        part = _sc_msg(h, e_flat, src, dst)
        h = _update(part, h, gW1[i], gb1[i], gW2[i], gb2[i], ln_w[i], ln_b[i])
    eC = _edge_enc(edge_attr, WC, bC)
    part = _sc_msg(h, e_flat, src, dst)
    hA, hB = _final_update(part, h, gW1[2], gb1[2], gW2[2], gb2[2],
                           ln_w[2], ln_b[2], WA, WB)

    t16 = _sc_edge_mlp(hA, hB, eC.reshape(E * H // 2), src, dst,
                       Wm2.reshape(H))
    return _lane_sum(t16, bm2)


# speedup vs baseline: 1.2998x; 1.2998x over previous
import functools

import jax
import jax.numpy as jnp
from jax import lax
from jax.experimental import pallas as pl
from jax.experimental.pallas import tpu as pltpu
from jax.experimental.pallas import tpu_sc as plsc

N = 10000
E = 320000
H = 128
DE = 16
NC = 2
NS = 16
NW = NC * NS
EPW = E // NW
BE = 4000
BL = 4000

C = 40
NCHUNKS = EPW // C
MSG_NBUF = 4
MSG_OUTER = NCHUNKS // MSG_NBUF
MSG_TAIL = NCHUNKS - MSG_OUTER * MSG_NBUF
NBLK = N // C

C2 = 40
NCH2 = EPW // C2
MLP_NBUF = 5
MLP_OUTER = NCH2 // MLP_NBUF

_mesh = plsc.VectorSubcoreMesh(core_axis_name="c", subcore_axis_name="s")


def _prep_body(x_ref, wn_ref, bn_ref, we_ref, wcm_ref, be_ref, bm1_ref,
               h_ref, wc_ref, bc_ref):
    h_ref[...] = jnp.dot(x_ref[...], wn_ref[...],
                         preferred_element_type=jnp.float32) + bn_ref[...]
    wc = jnp.dot(we_ref[...], wcm_ref[...], preferred_element_type=jnp.float32)
    wc_ref[...] = wc
    bc_ref[...] = jnp.dot(be_ref[...], wcm_ref[...],
                          preferred_element_type=jnp.float32) + bm1_ref[...]


def _prep(x, Wn, bn, We, Wcm, be, bm1):
    return pl.pallas_call(
        _prep_body,
        out_shape=(
            jax.ShapeDtypeStruct((N, H), jnp.float32),
            jax.ShapeDtypeStruct((DE, H), jnp.float32),
            jax.ShapeDtypeStruct((1, H), jnp.float32),
        ),
    )(x, Wn, bn.reshape(1, H), We, Wcm, be.reshape(1, H), bm1.reshape(1, H))


def _edge_enc_f32_body(ea_ref, w_ref, b_ref, e_ref):
    e_ref[...] = jnp.dot(ea_ref[...], w_ref[...],
                         preferred_element_type=jnp.float32) + b_ref[...]


def _edge_enc_f32(edge_attr, W, b):
    return pl.pallas_call(
        _edge_enc_f32_body,
        grid=(E // BE,),
        in_specs=[
            pl.BlockSpec((BE, DE), lambda i: (i, 0)),
            pl.BlockSpec((DE, H), lambda i: (0, 0)),
            pl.BlockSpec((1, H), lambda i: (0, 0)),
        ],
        out_specs=pl.BlockSpec((BE, H), lambda i: (i, 0)),
        out_shape=jax.ShapeDtypeStruct((E, H), jnp.float32),
    )(edge_attr, W, b)


def _edge_enc_body(ea_ref, w_ref, b_ref, e_ref):
    z = jnp.dot(ea_ref[...], w_ref[...],
                preferred_element_type=jnp.float32) + b_ref[...]
    u = jax.lax.bitcast_convert_type(z, jnp.uint32)
    b16 = (u + jnp.uint32(0x7FFF) + ((u >> 16) & jnp.uint32(1))) >> 16
    parts = []
    for g in range(H // 32):
        lo = b16[:, g * 32:g * 32 + 16]
        hi = b16[:, g * 32 + 16:g * 32 + 32]
        parts.append(lo | (hi << 16))
    e_ref[...] = jax.lax.bitcast_convert_type(
        jnp.concatenate(parts, axis=1), jnp.int32)


def _edge_enc(edge_attr, W, b):
    return pl.pallas_call(
        _edge_enc_body,
        grid=(E // BE,),
        in_specs=[
            pl.BlockSpec((BE, DE), lambda i: (i, 0)),
            pl.BlockSpec((DE, H), lambda i: (0, 0)),
            pl.BlockSpec((1, H), lambda i: (0, 0)),
        ],
        out_specs=pl.BlockSpec((BE, H // 2), lambda i: (i, 0)),
        out_shape=jax.ShapeDtypeStruct((E, H // 2), jnp.int32),
    )(edge_attr, W, b)


def _norm_mlp(p_ref, h_ref, w1_ref, b1_ref, w2_ref, b2_ref, lnw_ref, lnb_ref):
    agg = p_ref[0] + p_ref[1] + h_ref[...]
    z = jnp.maximum(jnp.dot(agg, w1_ref[...],
                            preferred_element_type=jnp.float32) + b1_ref[...], 0.0)
    z = jnp.dot(z, w2_ref[...], preferred_element_type=jnp.float32) + b2_ref[...]
    m = jnp.mean(z)
    v = jnp.mean((z - m) ** 2)
    z = (z - m) / (jnp.sqrt(v) + 1e-5)
    return jnp.maximum(z * lnw_ref[...] + lnb_ref[...], 0.0)


def _update_body(p_ref, h_ref, w1_ref, b1_ref, w2_ref, b2_ref, lnw_ref,
                 lnb_ref, o_ref):
    o_ref[...] = _norm_mlp(p_ref, h_ref, w1_ref, b1_ref, w2_ref, b2_ref,
                           lnw_ref, lnb_ref)


def _update(part, h, W1, b1, W2, b2, lnw, lnb):
    return pl.pallas_call(
        _update_body,
        out_shape=jax.ShapeDtypeStruct((N, H), jnp.float32),
    )(part, h, W1, b1.reshape(1, H), W2, b2.reshape(1, H),
      lnw.reshape(1, H), lnb.reshape(1, H))


def _final_update_body(p_ref, h_ref, w1_ref, b1_ref, w2_ref, b2_ref, lnw_ref,
                       lnb_ref, wa_ref, wb_ref, ha_ref, hb_ref):
    hn = _norm_mlp(p_ref, h_ref, w1_ref, b1_ref, w2_ref, b2_ref,
                   lnw_ref, lnb_ref)
    ha_ref[...] = jnp.dot(hn, wa_ref[...], preferred_element_type=jnp.float32)
    hb_ref[...] = jnp.dot(hn, wb_ref[...], preferred_element_type=jnp.float32)


def _final_update(part, h, W1, b1, W2, b2, lnw, lnb, WA, WB):
    return pl.pallas_call(
        _final_update_body,
        out_shape=(
            jax.ShapeDtypeStruct((N, H), jnp.float32),
            jax.ShapeDtypeStruct((N, H), jnp.float32),
        ),
    )(part, h, W1, b1.reshape(1, H), W2, b2.reshape(1, H),
      lnw.reshape(1, H), lnb.reshape(1, H), WA, WB)


def _lane_sum_body(t_ref, b2_ref, o_ref):
    j = lax.broadcasted_iota(jnp.int32, (128, 8), 0)
    g = lax.broadcasted_iota(jnp.int32, (128, 8), 1)
    m = (j // 16 == g).astype(jnp.float32)
    o_ref[...] = jnp.dot(t_ref[...], m,
                         preferred_element_type=jnp.float32) + b2_ref[0, 0]


def _lane_sum(t16_flat, bm2):
    t = t16_flat.reshape(E // 8, 128)
    out = pl.pallas_call(
        _lane_sum_body,
        grid=(E // 8 // BL,),
        in_specs=[
            pl.BlockSpec((BL, 128), lambda i: (i, 0)),
            pl.BlockSpec((1, 1), lambda i: (0, 0)),
        ],
        out_specs=pl.BlockSpec((BL, 8), lambda i: (i, 0)),
        out_shape=jax.ShapeDtypeStruct((E // 8, 8), jnp.float32),
    )(t, bm2.reshape(1, 1))
    return out.reshape(E)


@functools.partial(
    pl.kernel,
    out_type=jax.ShapeDtypeStruct((NC, N, H), jnp.float32),
    mesh=_mesh,
    compiler_params=pltpu.CompilerParams(needs_layout_passes=False),
    scratch_types=(
        [pltpu.VMEM((C,), jnp.int32) for _ in range(MSG_NBUF)]
        + [pltpu.VMEM((C,), jnp.int32) for _ in range(MSG_NBUF)]
        + [pltpu.VMEM((C, H), jnp.float32) for _ in range(MSG_NBUF)]
        + [pltpu.VMEM((C, H), jnp.float32) for _ in range(MSG_NBUF)]
        + [
            pltpu.VMEM_SHARED((N, H), jnp.float32),
            pltpu.SemaphoreType.DMA((MSG_NBUF,)),
            pltpu.SemaphoreType.DMA((MSG_NBUF,)),
            pltpu.SemaphoreType.DMA((MSG_NBUF,)),
            pltpu.SemaphoreType.DMA((MSG_NBUF,)),
            pltpu.SemaphoreType.DMA((MSG_NBUF,)),
        ]
    ),
)
def _sc_msg(h_hbm, e_hbm, src_hbm, dst_hbm, out_hbm, *refs):
    sidx = list(refs[0:MSG_NBUF])
    didx = list(refs[MSG_NBUF:2 * MSG_NBUF])
    gbuf = list(refs[2 * MSG_NBUF:3 * MSG_NBUF])
    ebuf = list(refs[3 * MSG_NBUF:4 * MSG_NBUF])
    agg, sem_si, sem_di, sem_g, sem_e, sem_sc = refs[4 * MSG_NBUF:]

    cid = lax.axis_index("c")
    sid = lax.axis_index("s")
    wid = cid * NS + sid
    base_w = wid * EPW

    zv = jnp.zeros((16,), jnp.float32)

    def zrow(r, carry):
        for k in range(H // 16):
            gbuf[0][r, pl.ds(k * 16, 16)] = zv
        return carry

    lax.fori_loop(0, C, zrow, 0)
    nblk = jnp.where(sid < NBLK - (NBLK // NS) * NS, NBLK // NS + 1, NBLK // NS)

    def zblk(j, carry):
        blk = sid + j * NS
        pltpu.sync_copy(gbuf[0], agg.at[pl.ds(blk * C, C)])
        return carry

    lax.fori_loop(0, nblk, zblk, 0)
    plsc.subcore_barrier()

    def compute_rows(gb, eb):
        def row(r, rc):
            for k in range(H // 16):
                g = gb[r, pl.ds(k * 16, 16)]
                ev = eb[r, pl.ds(k * 16, 16)]
                gb[r, pl.ds(k * 16, 16)] = jnp.maximum(g + ev, 0.0)
            return rc

        lax.fori_loop(0, C, row, 0)

    def outer(o, carry):
        i0 = o * MSG_NBUF

        @pl.when(o > 0)
        def _drain():
            for b in range(MSG_NBUF):
                pltpu.make_async_copy(h_hbm.at[pl.ds(0, C)], gbuf[b],
                                      sem_sc.at[b]).wait()

        dsi, ddi, dg, de = [], [], [], []
        for b in range(MSG_NBUF):
            eb = base_w + (i0 + b) * C
            dsi.append(pltpu.async_copy(src_hbm.at[pl.ds(eb, C)], sidx[b],
                                        sem_si.at[b]))
            ddi.append(pltpu.async_copy(dst_hbm.at[pl.ds(eb, C)], didx[b],
                                        sem_di.at[b]))
        for b in range(MSG_NBUF):
            eb = base_w + (i0 + b) * C
            dsi[b].wait()
            dg.append(pltpu.async_copy(h_hbm.at[sidx[b]], gbuf[b],
                                       sem_g.at[b]))
            de.append(pltpu.async_copy(e_hbm.at[pl.ds(eb, C)], ebuf[b],
                                       sem_e.at[b]))
        dsc = []
        for b in range(MSG_NBUF):
            dg[b].wait()
            de[b].wait()
            compute_rows(gbuf[b], ebuf[b])
            ddi[b].wait()
            dsc.append(pltpu.async_copy(gbuf[b], agg.at[didx[b]],
                                        sem_sc.at[b], add=True))
        return carry

    lax.fori_loop(0, MSG_OUTER, outer, 0)
    for b in range(MSG_NBUF):
        pltpu.make_async_copy(h_hbm.at[pl.ds(0, C)], gbuf[b],
                              sem_sc.at[b]).wait()

    for t in range(MSG_TAIL):
        i = MSG_OUTER * MSG_NBUF + t
        eb = base_w + i * C
        pltpu.sync_copy(src_hbm.at[pl.ds(eb, C)], sidx[t])
        pltpu.sync_copy(dst_hbm.at[pl.ds(eb, C)], didx[t])
        dg = pltpu.async_copy(h_hbm.at[sidx[t]], gbuf[t], sem_g.at[t])
        de = pltpu.async_copy(e_hbm.at[pl.ds(eb, C)], ebuf[t], sem_e.at[t])
        dg.wait()
        de.wait()
        compute_rows(gbuf[t], ebuf[t])
        pltpu.sync_copy(gbuf[t], agg.at[didx[t]], add=True)

    plsc.subcore_barrier()

    def oblk(j, carry):
        blk = sid + j * NS
        pltpu.sync_copy(agg.at[pl.ds(blk * C, C)],
                        out_hbm.at[cid, pl.ds(blk * C, C)])
        return carry

    lax.fori_loop(0, nblk, oblk, 0)


@functools.partial(
    pl.kernel,
    out_type=jax.ShapeDtypeStruct((E * 16,), jnp.float32),
    mesh=_mesh,
    compiler_params=pltpu.CompilerParams(needs_layout_passes=False),
    scratch_types=(
        [
            pltpu.VMEM((EPW,), jnp.int32),
            pltpu.VMEM((EPW,), jnp.int32),
            pltpu.VMEM((H,), jnp.float32),
        ]
        + [pltpu.VMEM((C2, H), jnp.float32) for _ in range(MLP_NBUF)]
        + [pltpu.VMEM((C2, H), jnp.float32) for _ in range(MLP_NBUF)]
        + [pltpu.VMEM((C2 * H // 2,), jnp.int32) for _ in range(MLP_NBUF)]
        + [pltpu.VMEM((C2 * 16,), jnp.float32) for _ in range(MLP_NBUF)]
        + [
            pltpu.SemaphoreType.DMA((MLP_NBUF,)),
            pltpu.SemaphoreType.DMA((MLP_NBUF,)),
            pltpu.SemaphoreType.DMA((MLP_NBUF,)),
            pltpu.SemaphoreType.DMA((MLP_NBUF,)),
        ]
    ),
)
def _sc_edge_mlp(ha_hbm, hb_hbm, ec_hbm, src_hbm, dst_hbm, w2_hbm, out_hbm,
                 *refs):
    sidx_all, didx_all, wbuf = refs[0:3]
    abuf = list(refs[3:3 + MLP_NBUF])
    bbuf = list(refs[3 + MLP_NBUF:3 + 2 * MLP_NBUF])
    cbuf = list(refs[3 + 2 * MLP_NBUF:3 + 3 * MLP_NBUF])
    obuf = list(refs[3 + 3 * MLP_NBUF:3 + 4 * MLP_NBUF])
    sem_a, sem_b, sem_c, sem_o = refs[3 + 4 * MLP_NBUF:]

    cid = lax.axis_index("c")
    sid = lax.axis_index("s")
    wid = cid * NS + sid
    base_w = wid * EPW

    pltpu.sync_copy(w2_hbm, wbuf)
    pltpu.sync_copy(src_hbm.at[pl.ds(base_w, EPW)], sidx_all)
    pltpu.sync_copy(dst_hbm.at[pl.ds(base_w, EPW)], didx_all)
    wv = [wbuf[pl.ds(k * 16, 16)] for k in range(H // 16)]

    def outer(o, carry):
        i0 = o * MLP_NBUF
        da, db, dc = [], [], []
        for b in range(MLP_NBUF):
            i = i0 + b
            da.append(pltpu.async_copy(
                ha_hbm.at[sidx_all.at[pl.ds(i * C2, C2)]], abuf[b],
                sem_a.at[b]))
            db.append(pltpu.async_copy(
                hb_hbm.at[didx_all.at[pl.ds(i * C2, C2)]], bbuf[b],
                sem_b.at[b]))
            dc.append(pltpu.async_copy(
                ec_hbm.at[pl.ds((base_w + i * C2) * (H // 2), C2 * H // 2)],
                cbuf[b], sem_c.at[b]))
        for b in range(MLP_NBUF):
            i = i0 + b

            @pl.when(o > 0)
            def _drain(_b=b):
                pltpu.make_async_copy(out_hbm.at[pl.ds(0, C2 * 16)],
                                      obuf[_b], sem_o.at[_b]).wait()

            da[b].wait()
            db[b].wait()
            dc[b].wait()

            def row2(r2, rc, _a=abuf[b], _b=bbuf[b], _c=cbuf[b],
                     _o=obuf[b]):
                for dr in range(2):
                    r = r2 * 2 + dr
                    acc = None
                    for k in range(H // 32):
                        cv = _c[pl.ds(r * (H // 2) + k * 16, 16)]
                        c0 = plsc.bitcast(cv << 16, jnp.float32)
                        c1 = plsc.bitcast(cv & jnp.int32(-65536), jnp.float32)
                        t0 = (_a[r, pl.ds(k * 32, 16)]
                              + _b[r, pl.ds(k * 32, 16)] + c0)
                        t0 = jnp.maximum(t0, 0.0) * wv[2 * k]
                        t1 = (_a[r, pl.ds(k * 32 + 16, 16)]
                              + _b[r, pl.ds(k * 32 + 16, 16)] + c1)
                        t1 = jnp.maximum(t1, 0.0) * wv[2 * k + 1]
                        acc = t0 + t1 if acc is None else acc + t0 + t1
                    _o[pl.ds(r * 16, 16)] = acc
                return rc

            lax.fori_loop(0, C2 // 2, row2, 0)
            pltpu.async_copy(
                obuf[b], out_hbm.at[pl.ds((base_w + i * C2) * 16, C2 * 16)],
                sem_o.at[b])
        return carry

    lax.fori_loop(0, MLP_OUTER, outer, 0)
    for b in range(MLP_NBUF):
        pltpu.make_async_copy(out_hbm.at[pl.ds(0, C2 * 16)], obuf[b],
                              sem_o.at[b]).wait()


def kernel(x, edge_index, edge_attr, Wn, bn, We, be, gW1, gb1, gW2, gb2,
           ln_w, ln_b, Wm1, bm1, Wm2, bm2):
    src = edge_index[0]
    dst = edge_index[1]
    WA = Wm1[:H]
    WB = Wm1[H:2 * H]

    WCm = Wm1[2 * H:]

    h, WC, bC = _prep(x, Wn, bn, We, WCm, be, bm1)
    e = _edge_enc_f32(edge_attr, We, be.reshape(1, H))

    for i in range(2):
        part = _sc_msg(h, e, src, dst)
        h = _update(part, h, gW1[i], gb1[i], gW2[i], gb2[i], ln_w[i], ln_b[i])
    eC = _edge_enc(edge_attr, WC, bC)
    part = _sc_msg(h, e, src, dst)
    hA, hB = _final_update(part, h, gW1[2], gb1[2], gW2[2], gb2[2],
                           ln_w[2], ln_b[2], WA, WB)

    t16 = _sc_edge_mlp(hA, hB, eC.reshape(E * H // 2), src, dst,
                       Wm2.reshape(H))
    return _lane_sum(t16, bm2)


# speedup vs baseline: 1.3465x; 1.0359x over previous
import functools

import jax
import jax.numpy as jnp
from jax import lax
from jax.experimental import pallas as pl
from jax.experimental.pallas import tpu as pltpu
from jax.experimental.pallas import tpu_sc as plsc

N = 10000
E = 320000
H = 128
DE = 16
NC = 2
NS = 16
NW = NC * NS
EPW = E // NW
BE = 4000
BL = 4000

C = 40
NCHUNKS = EPW // C
MSG_NBUF = 4
MSG_OUTER = NCHUNKS // MSG_NBUF
MSG_TAIL = NCHUNKS - MSG_OUTER * MSG_NBUF
NBLK = N // C

C2 = 40
NCH2 = EPW // C2
MLP_NBUF = 5
MLP_OUTER = NCH2 // MLP_NBUF

_mesh = plsc.VectorSubcoreMesh(core_axis_name="c", subcore_axis_name="s")


def _prep_body(x_ref, wn_ref, bn_ref, we_ref, wcm_ref, be_ref, bm1_ref,
               h_ref, wc_ref, bc_ref):
    h_ref[...] = jnp.dot(x_ref[...], wn_ref[...],
                         preferred_element_type=jnp.float32) + bn_ref[...]
    wc = jnp.dot(we_ref[...], wcm_ref[...], preferred_element_type=jnp.float32)
    wc_ref[...] = wc
    bc_ref[...] = jnp.dot(be_ref[...], wcm_ref[...],
                          preferred_element_type=jnp.float32) + bm1_ref[...]


def _prep(x, Wn, bn, We, Wcm, be, bm1):
    return pl.pallas_call(
        _prep_body,
        out_shape=(
            jax.ShapeDtypeStruct((N, H), jnp.float32),
            jax.ShapeDtypeStruct((DE, H), jnp.float32),
            jax.ShapeDtypeStruct((1, H), jnp.float32),
        ),
    )(x, Wn, bn.reshape(1, H), We, Wcm, be.reshape(1, H), bm1.reshape(1, H))


def _edge_enc_f32_body(ea_ref, w_ref, b_ref, e_ref):
    e_ref[...] = jnp.dot(ea_ref[...], w_ref[...],
                         preferred_element_type=jnp.float32) + b_ref[...]


def _edge_enc_f32(edge_attr, W, b):
    return pl.pallas_call(
        _edge_enc_f32_body,
        grid=(E // BE,),
        in_specs=[
            pl.BlockSpec((BE, DE), lambda i: (i, 0)),
            pl.BlockSpec((DE, H), lambda i: (0, 0)),
            pl.BlockSpec((1, H), lambda i: (0, 0)),
        ],
        out_specs=pl.BlockSpec((BE, H), lambda i: (i, 0)),
        out_shape=jax.ShapeDtypeStruct((E, H), jnp.float32),
    )(edge_attr, W, b)


def _edge_enc_body(ea_ref, w_ref, b_ref, e_ref):
    z = jnp.dot(ea_ref[...], w_ref[...],
                preferred_element_type=jnp.float32) + b_ref[...]
    u = jax.lax.bitcast_convert_type(z, jnp.uint32)
    b16 = (u + jnp.uint32(0x7FFF) + ((u >> 16) & jnp.uint32(1))) >> 16
    parts = []
    for g in range(H // 32):
        lo = b16[:, g * 32:g * 32 + 16]
        hi = b16[:, g * 32 + 16:g * 32 + 32]
        parts.append(lo | (hi << 16))
    e_ref[...] = jax.lax.bitcast_convert_type(
        jnp.concatenate(parts, axis=1), jnp.int32)


def _edge_enc(edge_attr, W, b):
    return pl.pallas_call(
        _edge_enc_body,
        grid=(E // BE,),
        in_specs=[
            pl.BlockSpec((BE, DE), lambda i: (i, 0)),
            pl.BlockSpec((DE, H), lambda i: (0, 0)),
            pl.BlockSpec((1, H), lambda i: (0, 0)),
        ],
        out_specs=pl.BlockSpec((BE, H // 2), lambda i: (i, 0)),
        out_shape=jax.ShapeDtypeStruct((E, H // 2), jnp.int32),
    )(edge_attr, W, b)


def _norm_mlp(p_ref, h_ref, w1_ref, b1_ref, w2_ref, b2_ref, lnw_ref, lnb_ref):
    agg = p_ref[0] + p_ref[1] + h_ref[...]
    z = jnp.maximum(jnp.dot(agg, w1_ref[...],
                            preferred_element_type=jnp.float32) + b1_ref[...], 0.0)
    z = jnp.dot(z, w2_ref[...], preferred_element_type=jnp.float32) + b2_ref[...]
    m = jnp.mean(z)
    v = jnp.mean((z - m) ** 2)
    z = (z - m) / (jnp.sqrt(v) + 1e-5)
    return jnp.maximum(z * lnw_ref[...] + lnb_ref[...], 0.0)


def _update_body(p_ref, h_ref, w1_ref, b1_ref, w2_ref, b2_ref, lnw_ref,
                 lnb_ref, o_ref):
    o_ref[...] = _norm_mlp(p_ref, h_ref, w1_ref, b1_ref, w2_ref, b2_ref,
                           lnw_ref, lnb_ref)


def _update(part, h, W1, b1, W2, b2, lnw, lnb):
    return pl.pallas_call(
        _update_body,
        out_shape=jax.ShapeDtypeStruct((N, H), jnp.float32),
    )(part, h, W1, b1.reshape(1, H), W2, b2.reshape(1, H),
      lnw.reshape(1, H), lnb.reshape(1, H))


def _final_update_body(p_ref, h_ref, w1_ref, b1_ref, w2_ref, b2_ref, lnw_ref,
                       lnb_ref, wa_ref, wb_ref, ha_ref, hb_ref):
    hn = _norm_mlp(p_ref, h_ref, w1_ref, b1_ref, w2_ref, b2_ref,
                   lnw_ref, lnb_ref)
    ha_ref[...] = jnp.dot(hn, wa_ref[...], preferred_element_type=jnp.float32)
    hb_ref[...] = jnp.dot(hn, wb_ref[...], preferred_element_type=jnp.float32)


def _final_update(part, h, W1, b1, W2, b2, lnw, lnb, WA, WB):
    return pl.pallas_call(
        _final_update_body,
        out_shape=(
            jax.ShapeDtypeStruct((N, H), jnp.float32),
            jax.ShapeDtypeStruct((N, H), jnp.float32),
        ),
    )(part, h, W1, b1.reshape(1, H), W2, b2.reshape(1, H),
      lnw.reshape(1, H), lnb.reshape(1, H), WA, WB)


def _lane_sum_body(t_ref, b2_ref, o_ref):
    j = lax.broadcasted_iota(jnp.int32, (128, 8), 0)
    g = lax.broadcasted_iota(jnp.int32, (128, 8), 1)
    m = (j // 16 == g).astype(jnp.float32)
    o_ref[...] = jnp.dot(t_ref[...], m,
                         preferred_element_type=jnp.float32) + b2_ref[0, 0]


def _lane_sum(t16_flat, bm2):
    t = t16_flat.reshape(E // 8, 128)
    out = pl.pallas_call(
        _lane_sum_body,
        grid=(E // 8 // BL,),
        in_specs=[
            pl.BlockSpec((BL, 128), lambda i: (i, 0)),
            pl.BlockSpec((1, 1), lambda i: (0, 0)),
        ],
        out_specs=pl.BlockSpec((BL, 8), lambda i: (i, 0)),
        out_shape=jax.ShapeDtypeStruct((E // 8, 8), jnp.float32),
    )(t, bm2.reshape(1, 1))
    return out.reshape(E)


@functools.partial(
    pl.kernel,
    out_type=jax.ShapeDtypeStruct((NC, N, H), jnp.float32),
    mesh=_mesh,
    compiler_params=pltpu.CompilerParams(needs_layout_passes=False),
    scratch_types=(
        [pltpu.VMEM((C,), jnp.int32) for _ in range(MSG_NBUF)]
        + [pltpu.VMEM((C,), jnp.int32) for _ in range(MSG_NBUF)]
        + [pltpu.VMEM((C, H), jnp.float32) for _ in range(MSG_NBUF)]
        + [pltpu.VMEM((C, H), jnp.float32) for _ in range(MSG_NBUF)]
        + [
            pltpu.VMEM_SHARED((N, H), jnp.float32),
            pltpu.SemaphoreType.DMA((MSG_NBUF,)),
            pltpu.SemaphoreType.DMA((MSG_NBUF,)),
            pltpu.SemaphoreType.DMA((MSG_NBUF,)),
            pltpu.SemaphoreType.DMA((MSG_NBUF,)),
            pltpu.SemaphoreType.DMA((MSG_NBUF,)),
        ]
    ),
)
def _sc_msg(h_hbm, e_hbm, src_hbm, dst_hbm, out_hbm, *refs):
    sidx = list(refs[0:MSG_NBUF])
    didx = list(refs[MSG_NBUF:2 * MSG_NBUF])
    gbuf = list(refs[2 * MSG_NBUF:3 * MSG_NBUF])
    ebuf = list(refs[3 * MSG_NBUF:4 * MSG_NBUF])
    agg, sem_si, sem_di, sem_g, sem_e, sem_sc = refs[4 * MSG_NBUF:]

    cid = lax.axis_index("c")
    sid = lax.axis_index("s")
    wid = cid * NS + sid
    base_w = wid * EPW

    zv = jnp.zeros((16,), jnp.float32)

    def zrow(r, carry):
        for k in range(H // 16):
            gbuf[0][r, pl.ds(k * 16, 16)] = zv
        return carry

    lax.fori_loop(0, C, zrow, 0)
    nblk = jnp.where(sid < NBLK - (NBLK // NS) * NS, NBLK // NS + 1, NBLK // NS)

    def zblk(j, carry):
        blk = sid + j * NS
        pltpu.sync_copy(gbuf[0], agg.at[pl.ds(blk * C, C)])
        return carry

    lax.fori_loop(0, nblk, zblk, 0)
    plsc.subcore_barrier()

    def compute_rows(gb, eb):
        @plsc.parallel_loop(0, C, step=1, unroll=4)
        def _row(r):
            for k in range(H // 16):
                g = gb[r, pl.ds(k * 16, 16)]
                ev = eb[r, pl.ds(k * 16, 16)]
                gb[r, pl.ds(k * 16, 16)] = jnp.maximum(g + ev, 0.0)

    def outer(o, carry):
        i0 = o * MSG_NBUF

        @pl.when(o > 0)
        def _drain():
            for b in range(MSG_NBUF):
                pltpu.make_async_copy(h_hbm.at[pl.ds(0, C)], gbuf[b],
                                      sem_sc.at[b]).wait()

        dsi, ddi, dg, de = [], [], [], []
        for b in range(MSG_NBUF):
            eb = base_w + (i0 + b) * C
            dsi.append(pltpu.async_copy(src_hbm.at[pl.ds(eb, C)], sidx[b],
                                        sem_si.at[b]))
            ddi.append(pltpu.async_copy(dst_hbm.at[pl.ds(eb, C)], didx[b],
                                        sem_di.at[b]))
        for b in range(MSG_NBUF):
            eb = base_w + (i0 + b) * C
            dsi[b].wait()
            dg.append(pltpu.async_copy(h_hbm.at[sidx[b]], gbuf[b],
                                       sem_g.at[b]))
            de.append(pltpu.async_copy(e_hbm.at[pl.ds(eb, C)], ebuf[b],
                                       sem_e.at[b]))
        dsc = []
        for b in range(MSG_NBUF):
            dg[b].wait()
            de[b].wait()
            compute_rows(gbuf[b], ebuf[b])
            ddi[b].wait()
            dsc.append(pltpu.async_copy(gbuf[b], agg.at[didx[b]],
                                        sem_sc.at[b], add=True))
        return carry

    lax.fori_loop(0, MSG_OUTER, outer, 0)
    for b in range(MSG_NBUF):
        pltpu.make_async_copy(h_hbm.at[pl.ds(0, C)], gbuf[b],
                              sem_sc.at[b]).wait()

    for t in range(MSG_TAIL):
        i = MSG_OUTER * MSG_NBUF + t
        eb = base_w + i * C
        pltpu.sync_copy(src_hbm.at[pl.ds(eb, C)], sidx[t])
        pltpu.sync_copy(dst_hbm.at[pl.ds(eb, C)], didx[t])
        dg = pltpu.async_copy(h_hbm.at[sidx[t]], gbuf[t], sem_g.at[t])
        de = pltpu.async_copy(e_hbm.at[pl.ds(eb, C)], ebuf[t], sem_e.at[t])
        dg.wait()
        de.wait()
        compute_rows(gbuf[t], ebuf[t])
        pltpu.sync_copy(gbuf[t], agg.at[didx[t]], add=True)

    plsc.subcore_barrier()

    def oblk(j, carry):
        blk = sid + j * NS
        pltpu.sync_copy(agg.at[pl.ds(blk * C, C)],
                        out_hbm.at[cid, pl.ds(blk * C, C)])
        return carry

    lax.fori_loop(0, nblk, oblk, 0)


@functools.partial(
    pl.kernel,
    out_type=jax.ShapeDtypeStruct((E * 16,), jnp.float32),
    mesh=_mesh,
    compiler_params=pltpu.CompilerParams(needs_layout_passes=False),
    scratch_types=(
        [
            pltpu.VMEM((EPW,), jnp.int32),
            pltpu.VMEM((EPW,), jnp.int32),
            pltpu.VMEM((H,), jnp.float32),
        ]
        + [pltpu.VMEM((C2, H), jnp.float32) for _ in range(MLP_NBUF)]
        + [pltpu.VMEM((C2, H), jnp.float32) for _ in range(MLP_NBUF)]
        + [pltpu.VMEM((C2 * H // 2,), jnp.int32) for _ in range(MLP_NBUF)]
        + [pltpu.VMEM((C2 * 16,), jnp.float32) for _ in range(MLP_NBUF)]
        + [
            pltpu.SemaphoreType.DMA((MLP_NBUF,)),
            pltpu.SemaphoreType.DMA((MLP_NBUF,)),
            pltpu.SemaphoreType.DMA((MLP_NBUF,)),
            pltpu.SemaphoreType.DMA((MLP_NBUF,)),
        ]
    ),
)
def _sc_edge_mlp(ha_hbm, hb_hbm, ec_hbm, src_hbm, dst_hbm, w2_hbm, out_hbm,
                 *refs):
    sidx_all, didx_all, wbuf = refs[0:3]
    abuf = list(refs[3:3 + MLP_NBUF])
    bbuf = list(refs[3 + MLP_NBUF:3 + 2 * MLP_NBUF])
    cbuf = list(refs[3 + 2 * MLP_NBUF:3 + 3 * MLP_NBUF])
    obuf = list(refs[3 + 3 * MLP_NBUF:3 + 4 * MLP_NBUF])
    sem_a, sem_b, sem_c, sem_o = refs[3 + 4 * MLP_NBUF:]

    cid = lax.axis_index("c")
    sid = lax.axis_index("s")
    wid = cid * NS + sid
    base_w = wid * EPW

    pltpu.sync_copy(w2_hbm, wbuf)
    pltpu.sync_copy(src_hbm.at[pl.ds(base_w, EPW)], sidx_all)
    pltpu.sync_copy(dst_hbm.at[pl.ds(base_w, EPW)], didx_all)
    wv = [wbuf[pl.ds(k * 16, 16)] for k in range(H // 16)]

    def outer(o, carry):
        i0 = o * MLP_NBUF
        da, db, dc = [], [], []
        for b in range(MLP_NBUF):
            i = i0 + b
            da.append(pltpu.async_copy(
                ha_hbm.at[sidx_all.at[pl.ds(i * C2, C2)]], abuf[b],
                sem_a.at[b]))
            db.append(pltpu.async_copy(
                hb_hbm.at[didx_all.at[pl.ds(i * C2, C2)]], bbuf[b],
                sem_b.at[b]))
            dc.append(pltpu.async_copy(
                ec_hbm.at[pl.ds((base_w + i * C2) * (H // 2), C2 * H // 2)],
                cbuf[b], sem_c.at[b]))
        for b in range(MLP_NBUF):
            i = i0 + b

            @pl.when(o > 0)
            def _drain(_b=b):
                pltpu.make_async_copy(out_hbm.at[pl.ds(0, C2 * 16)],
                                      obuf[_b], sem_o.at[_b]).wait()

            da[b].wait()
            db[b].wait()
            dc[b].wait()

            _a, _b, _c, _o = abuf[b], bbuf[b], cbuf[b], obuf[b]

            @plsc.parallel_loop(0, C2, step=1, unroll=2)
            def _row(r):
                acc = None
                for k in range(H // 32):
                    cv = _c[pl.ds(r * (H // 2) + k * 16, 16)]
                    c0 = plsc.bitcast(cv << 16, jnp.float32)
                    c1 = plsc.bitcast(cv & jnp.int32(-65536), jnp.float32)
                    t0 = (_a[r, pl.ds(k * 32, 16)]
                          + _b[r, pl.ds(k * 32, 16)] + c0)
                    t0 = jnp.maximum(t0, 0.0) * wv[2 * k]
                    t1 = (_a[r, pl.ds(k * 32 + 16, 16)]
                          + _b[r, pl.ds(k * 32 + 16, 16)] + c1)
                    t1 = jnp.maximum(t1, 0.0) * wv[2 * k + 1]
                    acc = t0 + t1 if acc is None else acc + t0 + t1
                _o[pl.ds(r * 16, 16)] = acc
            pltpu.async_copy(
                obuf[b], out_hbm.at[pl.ds((base_w + i * C2) * 16, C2 * 16)],
                sem_o.at[b])
        return carry

    lax.fori_loop(0, MLP_OUTER, outer, 0)
    for b in range(MLP_NBUF):
        pltpu.make_async_copy(out_hbm.at[pl.ds(0, C2 * 16)], obuf[b],
                              sem_o.at[b]).wait()


def kernel(x, edge_index, edge_attr, Wn, bn, We, be, gW1, gb1, gW2, gb2,
           ln_w, ln_b, Wm1, bm1, Wm2, bm2):
    src = edge_index[0]
    dst = edge_index[1]
    WA = Wm1[:H]
    WB = Wm1[H:2 * H]

    WCm = Wm1[2 * H:]

    h, WC, bC = _prep(x, Wn, bn, We, WCm, be, bm1)
    e = _edge_enc_f32(edge_attr, We, be.reshape(1, H))

    for i in range(2):
        part = _sc_msg(h, e, src, dst)
        h = _update(part, h, gW1[i], gb1[i], gW2[i], gb2[i], ln_w[i], ln_b[i])
    eC = _edge_enc(edge_attr, WC, bC)
    part = _sc_msg(h, e, src, dst)
    hA, hB = _final_update(part, h, gW1[2], gb1[2], gW2[2], gb2[2],
                           ln_w[2], ln_b[2], WA, WB)

    t16 = _sc_edge_mlp(hA, hB, eC.reshape(E * H // 2), src, dst,
                       Wm2.reshape(H))
    return _lane_sum(t16, bm2)
